# baseline (speedup 1.0000x reference)
import functools

import jax
import jax.numpy as jnp
from jax import lax
from jax.experimental import pallas as pl
from jax.experimental.pallas import tpu as pltpu
from jax.experimental.pallas import tpu_sc as plsc

N = 16384
D = 64

_NC = 2
_NS = 16
_NW = _NC * _NS
_RPW = N // _NW
_CHUNK = 64
_NCH = _RPW // _CHUNK

_mesh = plsc.VectorSubcoreMesh(core_axis_name="c", subcore_axis_name="s")


@functools.partial(
    pl.kernel,
    mesh=_mesh,
    compiler_params=pltpu.CompilerParams(needs_layout_passes=False),
    out_type=jax.ShapeDtypeStruct((N,), jnp.float32),
    scratch_types=[
        pltpu.VMEM((_RPW,), jnp.int32),
        pltpu.VMEM((_RPW,), jnp.int32),
        pltpu.VMEM((_RPW,), jnp.int32),
        pltpu.VMEM((D, _RPW), jnp.float32),
        pltpu.VMEM((D * 16,), jnp.float32),
        pltpu.VMEM((_RPW,), jnp.float32),
        pltpu.VMEM((_RPW,), jnp.float32),
        pltpu.VMEM((_RPW,), jnp.float32),
        pltpu.VMEM((1024,), jnp.float32),
        pltpu.SemaphoreType.DMA,
        pltpu.SemaphoreType.DMA,
        pltpu.SemaphoreType.DMA,
    ],
)
def _sc_fused(xt_hbm, pro_hbm, celeb_hbm, season_hbm, beta_hbm, u_hbm, v_hbm,
              w_hbm, out_hbm, idu, idv, ids, xcol, bbv, acc, gu, gv, wtab,
              sem_i, sem_x, sem_g):
    wid = lax.axis_index("s") * _NC + lax.axis_index("c")
    base = wid * _RPW

    stage = [
        pltpu.async_copy(pro_hbm.at[pl.ds(base, _RPW)], idu, sem_i),
        pltpu.async_copy(celeb_hbm.at[pl.ds(base, _RPW)], idv, sem_i),
        pltpu.async_copy(season_hbm.at[pl.ds(base, _RPW)], ids, sem_i),
        pltpu.async_copy(beta_hbm, bbv, sem_i),
        pltpu.async_copy(w_hbm, wtab.at[pl.ds(0, 1000)], sem_i),
    ]
    _DQ = D // 4
    xcps = [
        pltpu.async_copy(
            xt_hbm.at[pl.ds(q * _DQ, _DQ), pl.ds(base, _RPW)],
            xcol.at[pl.ds(q * _DQ, _DQ), :], sem_x)
        for q in range(4)
    ]
    for c in stage:
        c.wait()

    def fire_body(j, _):
        sl = pl.ds(j * _CHUNK, _CHUNK)
        pltpu.async_copy(u_hbm.at[idu.at[sl]], gu.at[sl], sem_g)
        pltpu.async_copy(v_hbm.at[idv.at[sl]], gv.at[sl], sem_g)
        return _

    lax.fori_loop(0, _NCH, fire_body, 0)

    for q in range(4):
        xcps[q].wait()

        def chunk_body(c, _, q=q):
            r1 = pl.ds(c * 32, 16)
            r2 = pl.ds(c * 32 + 16, 16)
            d0 = q * _DQ
            if q:
                a1 = acc[r1]
                a2 = acc[r2]
            else:
                b = bbv[pl.ds(0, 16)]
                a1 = xcol[0, r1] * b
                a2 = xcol[0, r2] * b
                d0 = 1
            for d in range(d0, (q + 1) * _DQ):
                b = bbv[pl.ds(d * 16, 16)]
                a1 = a1 + xcol[d, r1] * b
                a2 = a2 + xcol[d, r2] * b
            acc[r1] = a1
            acc[r2] = a2
            return _

        lax.fori_loop(0, _RPW // 32, chunk_body, 0)

    pltpu.make_async_copy(u_hbm.at[pl.ds(0, _RPW)], gu, sem_g).wait()
    pltpu.make_async_copy(v_hbm.at[pl.ds(0, _RPW)], gv, sem_g).wait()

    def add_body(i, _):
        s = pl.ds(i * 16, 16)
        acc[s] = acc[s] + gu[s] + gv[s] + plsc.load_gather(wtab, [ids[s]])
        return _

    lax.fori_loop(0, _RPW // 16, add_body, 0)

    pltpu.sync_copy(acc, out_hbm.at[pl.ds(base, _RPW)])


def kernel(X, pro_id, celeb_id, season, beta, u_pro, v_celeb, w_season):
    return _sc_fused(
        X.T,
        pro_id.astype(jnp.int32),
        celeb_id.astype(jnp.int32),
        season.astype(jnp.int32),
        jnp.repeat(beta, 16),
        u_pro,
        v_celeb,
        w_season,
    )


# speedup vs baseline: 1.0086x; 1.0086x over previous
import functools

import jax
import jax.numpy as jnp
from jax import lax
from jax.experimental import pallas as pl
from jax.experimental.pallas import tpu as pltpu
from jax.experimental.pallas import tpu_sc as plsc

N = 16384
D = 64

_NC = 2
_NS = 16
_NW = _NC * _NS
_RPW = N // _NW
_CHUNK = 64
_NCH = _RPW // _CHUNK

_mesh = plsc.VectorSubcoreMesh(core_axis_name="c", subcore_axis_name="s")


@functools.partial(
    pl.kernel,
    mesh=_mesh,
    compiler_params=pltpu.CompilerParams(needs_layout_passes=False),
    out_type=jax.ShapeDtypeStruct((N,), jnp.float32),
    scratch_types=[
        pltpu.VMEM((_RPW,), jnp.int32),
        pltpu.VMEM((_RPW,), jnp.int32),
        pltpu.VMEM((_RPW,), jnp.int32),
        pltpu.VMEM((D, _RPW), jnp.float32),
        pltpu.VMEM((D * 16,), jnp.float32),
        pltpu.VMEM((_RPW,), jnp.float32),
        pltpu.VMEM((_RPW,), jnp.float32),
        pltpu.VMEM((_RPW,), jnp.float32),
        pltpu.VMEM((1024,), jnp.float32),
        pltpu.SemaphoreType.DMA,
        pltpu.SemaphoreType.DMA,
        pltpu.SemaphoreType.DMA,
    ],
)
def _sc_fused(xt_hbm, pro_hbm, celeb_hbm, season_hbm, beta_hbm, u_hbm, v_hbm,
              w_hbm, out_hbm, idu, idv, ids, xcol, bbv, acc, gu, gv, wtab,
              sem_i, sem_x, sem_g):
    wid = lax.axis_index("s") * _NC + lax.axis_index("c")
    base = wid * _RPW

    stage = [
        pltpu.async_copy(pro_hbm.at[pl.ds(base, _RPW)], idu, sem_i),
        pltpu.async_copy(celeb_hbm.at[pl.ds(base, _RPW)], idv, sem_i),
        pltpu.async_copy(season_hbm.at[pl.ds(base, _RPW)], ids, sem_i),
        pltpu.async_copy(beta_hbm, bbv, sem_i),
        pltpu.async_copy(w_hbm, wtab.at[pl.ds(0, 1000)], sem_i),
    ]
    xcps = [
        pltpu.async_copy(
            xt_hbm.at[pl.ds(h * (D // 2), D // 2), pl.ds(base, _RPW)],
            xcol.at[pl.ds(h * (D // 2), D // 2), :], sem_x)
        for h in range(2)
    ]
    for c in stage:
        c.wait()

    def fire_body(j, _):
        sl = pl.ds(j * _CHUNK, _CHUNK)
        pltpu.async_copy(u_hbm.at[idu.at[sl]], gu.at[sl], sem_g)
        pltpu.async_copy(v_hbm.at[idv.at[sl]], gv.at[sl], sem_g)
        return _

    lax.fori_loop(0, _NCH, fire_body, 0)

    for h in range(2):
        xcps[h].wait()

        def chunk_body(c, _, h=h):
            r1 = pl.ds(c * 32, 16)
            r2 = pl.ds(c * 32 + 16, 16)
            d0 = h * (D // 2)
            if h:
                a1 = acc[r1]
                a2 = acc[r2]
            else:
                b = bbv[pl.ds(0, 16)]
                a1 = xcol[0, r1] * b
                a2 = xcol[0, r2] * b
                d0 = 1
            for d in range(d0, (h + 1) * (D // 2)):
                b = bbv[pl.ds(d * 16, 16)]
                a1 = a1 + xcol[d, r1] * b
                a2 = a2 + xcol[d, r2] * b
            acc[r1] = a1
            acc[r2] = a2
            return _

        lax.fori_loop(0, _RPW // 32, chunk_body, 0)

    pltpu.make_async_copy(u_hbm.at[pl.ds(0, _RPW)], gu, sem_g).wait()
    pltpu.make_async_copy(v_hbm.at[pl.ds(0, _RPW)], gv, sem_g).wait()

    def add_body(i, _):
        s = pl.ds(i * 16, 16)
        acc[s] = acc[s] + gu[s] + gv[s] + plsc.load_gather(wtab, [ids[s]])
        return _

    lax.fori_loop(0, _RPW // 16, add_body, 0)

    pltpu.sync_copy(acc, out_hbm.at[pl.ds(base, _RPW)])


def kernel(X, pro_id, celeb_id, season, beta, u_pro, v_celeb, w_season):
    return _sc_fused(
        X.T,
        pro_id.astype(jnp.int32),
        celeb_id.astype(jnp.int32),
        season.astype(jnp.int32),
        jnp.repeat(beta, 16),
        u_pro,
        v_celeb,
        w_season,
    )
